# Initial kernel scaffold; baseline (speedup 1.0000x reference)
import functools

import numpy as np
import jax
import jax.numpy as jnp
from jax import lax
from jax.experimental import pallas as pl
from jax.experimental.pallas import tpu as pltpu
from jax.experimental.pallas import tpu_sc as plsc

L = 16
NC, NS = 2, 16
NW = NC * NS


def _positional_encoding(seq_len, d_model):
    pos = np.arange(seq_len)[:, np.newaxis]
    dim = np.arange(d_model)[np.newaxis, :]
    angles = pos / np.power(10000, 2 * (dim // 2) / d_model)
    pe = np.zeros(angles.shape)
    pe[:, 0::2] = np.sin(angles[:, 0::2])
    pe[:, 1::2] = np.cos(angles[:, 1::2])
    return pe.astype(np.float32)


@functools.partial(jax.jit, static_argnums=(3, 4))
def _embed(idx_flat, table, pe, seq, d):
    ntot = idx_flat.shape[0]
    per_w = ntot // NW
    bpc = 2
    rows = bpc * seq
    nchunks = per_w // rows
    seg = 80
    nseg = rows // seg
    ngrp = rows // L

    mesh = plsc.VectorSubcoreMesh(core_axis_name="c", subcore_axis_name="s")

    @functools.partial(
        pl.kernel,
        mesh=mesh,
        out_type=jax.ShapeDtypeStruct((ntot, d), jnp.float32),
        scratch_types=[
            pltpu.VMEM((rows,), jnp.int32),
            pltpu.VMEM((rows, d), jnp.float32),
            pltpu.VMEM((rows, d), jnp.float32),
            pltpu.SemaphoreType.DMA,
        ],
    )
    def body(idx_hbm, table_hbm, pe_hbm, out_hbm, idx_v, rows_v, pe_v, sem):
        wid = lax.axis_index("s") * NC + lax.axis_index("c")
        base0 = wid * per_w

        for b in range(bpc):
            pltpu.sync_copy(pe_hbm, pe_v.at[pl.ds(b * seq, seq)])

        def chunk(ci, _):
            base = base0 + ci * rows
            pltpu.sync_copy(idx_hbm.at[pl.ds(base, rows)], idx_v)
            descs = [
                pltpu.async_copy(
                    table_hbm.at[idx_v.at[pl.ds(g * seg, seg)]],
                    rows_v.at[pl.ds(g * seg, seg)],
                    sem,
                )
                for g in range(nseg)
            ]
            for dsc in descs:
                dsc.wait()

            def fix(g, _):
                v = idx_v[pl.ds(g * L, L)]
                m = v == 0

                @pl.when(jnp.any(m))
                def _():
                    row16 = g * L + lax.broadcasted_iota(jnp.int32, (L,), 0)
                    zeros = jnp.zeros((L,), jnp.float32)
                    for j in range(d):
                        col16 = jnp.full((L,), j, jnp.int32)
                        plsc.store_scatter(rows_v, [row16, col16], zeros, mask=m)

                return 0

            lax.fori_loop(0, ngrp, fix, 0)

            def add_pe(r, _):
                for q in range(d // L):
                    plsc.addupdate(
                        rows_v.at[r, pl.ds(q * L, L)],
                        pe_v[r, pl.ds(q * L, L)],
                    )
                return 0

            lax.fori_loop(0, rows, add_pe, 0)

            pltpu.sync_copy(rows_v, out_hbm.at[pl.ds(base, rows)])
            return 0

        lax.fori_loop(0, nchunks, chunk, 0)

    return body(idx_flat, table, pe)


def kernel(input, table):
    b, s = input.shape
    v, d = table.shape
    idx_flat = input.reshape(-1).astype(jnp.int32)
    pe = jnp.asarray(_positional_encoding(s, d))
    out = _embed(idx_flat, table, pe, s, d)
    return out.reshape(b, s, d)


# speedup vs baseline: 3.2480x; 3.2480x over previous
import functools

import numpy as np
import jax
import jax.numpy as jnp
from jax import lax
from jax.experimental import pallas as pl
from jax.experimental.pallas import tpu as pltpu
from jax.experimental.pallas import tpu_sc as plsc

L = 16
NC, NS = 2, 16
NW = NC * NS


def _positional_encoding(seq_len, d_model):
    pos = np.arange(seq_len)[:, np.newaxis]
    dim = np.arange(d_model)[np.newaxis, :]
    angles = pos / np.power(10000, 2 * (dim // 2) / d_model)
    pe = np.zeros(angles.shape)
    pe[:, 0::2] = np.sin(angles[:, 0::2])
    pe[:, 1::2] = np.cos(angles[:, 1::2])
    return pe.astype(np.float32)


@functools.partial(jax.jit, static_argnums=(3, 4))
def _embed(idx_flat, table, pe, seq, d):
    ntot = idx_flat.shape[0]
    per_w = ntot // NW
    bpc = 2
    rows = bpc * seq
    nchunks = per_w // rows
    seg = 80
    nseg = rows // seg
    ngrp = rows // L

    mesh = plsc.VectorSubcoreMesh(core_axis_name="c", subcore_axis_name="s")

    @functools.partial(
        pl.kernel,
        mesh=mesh,
        compiler_params=pltpu.CompilerParams(
            needs_layout_passes=False, use_tc_tiling_on_sc=False
        ),
        out_type=jax.ShapeDtypeStruct((ntot, d), jnp.float32),
        scratch_types=[
            pltpu.VMEM((rows,), jnp.int32),
            pltpu.VMEM((rows, d), jnp.float32),
            pltpu.VMEM((rows, d), jnp.float32),
            pltpu.SemaphoreType.DMA,
        ],
    )
    def body(idx_hbm, table_hbm, pe_hbm, out_hbm, idx_v, rows_v, pe_v, sem):
        wid = lax.axis_index("s") * NC + lax.axis_index("c")
        base0 = wid * per_w

        for b in range(bpc):
            pltpu.sync_copy(pe_hbm, pe_v.at[pl.ds(b * seq, seq)])

        def chunk(ci, _):
            base = base0 + ci * rows
            pltpu.sync_copy(idx_hbm.at[pl.ds(base, rows)], idx_v)
            descs = [
                pltpu.async_copy(
                    table_hbm.at[idx_v.at[pl.ds(g * seg, seg)]],
                    rows_v.at[pl.ds(g * seg, seg)],
                    sem,
                )
                for g in range(nseg)
            ]
            for dsc in descs:
                dsc.wait()

            def fix(g, _):
                v = idx_v[pl.ds(g * L, L)]
                m = v == 0
                npad = plsc.all_reduce_population_count(m)[0]

                @pl.when(npad > 0)
                def _():
                    row16 = g * L + lax.broadcasted_iota(jnp.int32, (L,), 0)
                    zeros = jnp.zeros((L,), jnp.float32)
                    for j in range(d):
                        col16 = jnp.full((L,), j, jnp.int32)
                        plsc.store_scatter(rows_v, [row16, col16], zeros, mask=m)

                return 0

            lax.fori_loop(0, ngrp, fix, 0)

            def add_pe(r, _):
                for q in range(d // L):
                    plsc.addupdate(
                        rows_v.at[r, pl.ds(q * L, L)],
                        pe_v[r, pl.ds(q * L, L)],
                    )
                return 0

            lax.fori_loop(0, rows, add_pe, 0)

            pltpu.sync_copy(rows_v, out_hbm.at[pl.ds(base, rows)])
            return 0

        lax.fori_loop(0, nchunks, chunk, 0)

    return body(idx_flat, table, pe)


def kernel(input, table):
    b, s = input.shape
    v, d = table.shape
    idx_flat = input.reshape(-1).astype(jnp.int32)
    pe = jnp.asarray(_positional_encoding(s, d))
    out = _embed(idx_flat, table, pe, s, d)
    return out.reshape(b, s, d)
